# baseline (speedup 1.0000x reference)
import functools

import numpy as np

import jax
import jax.numpy as jnp
from jax import lax
from jax.experimental import pallas as pl
from jax.experimental.pallas import tpu as pltpu
from jax.experimental.pallas import tpu_sc as plsc

N = 10000
NP = 10240
E = 320000
EP = 327680
EB = EP // 128
D = 128
OUT = 64
NC = 2
NS = 16
RPT = NP // NS
BPT = EB // (NC * NS)
HB = BPT // 2

_SC_MESH = dict(core_axis_name="c", subcore_axis_name="s",
                num_cores=NC, num_subcores=NS)


@functools.partial(
    pl.kernel,
    out_type=jax.ShapeDtypeStruct((NC, NP), jnp.float32),
    mesh=plsc.VectorSubcoreMesh(**_SC_MESH),
    scratch_types=[
        pltpu.VMEM((BPT, 128), jnp.int32),
        pltpu.VMEM((128,), jnp.float32),
        pltpu.VMEM((RPT,), jnp.float32),
        pltpu.VMEM_SHARED((NP,), jnp.float32),
        pltpu.SemaphoreType.DMA,
    ],
)
def _sc_degree(ei_hbm, deg_out, dst_buf, ones_buf, row_buf, deg_shared, sem):
    c = lax.axis_index("c")
    s = lax.axis_index("s")
    ones16 = jnp.ones((16,), jnp.float32)
    zeros16 = jnp.zeros((16,), jnp.float32)
    for i in range(8):
        ones_buf[pl.ds(i * 16, 16)] = ones16

    def zbody(i, _):
        row_buf[pl.ds(i * 16, 16)] = zeros16
        return 0
    lax.fori_loop(0, RPT // 16, zbody, 0)
    pltpu.sync_copy(row_buf, deg_shared.at[pl.ds(s * RPT, RPT)])

    b0 = c * (EB // NC) + s * BPT
    pltpu.sync_copy(ei_hbm.at[1, pl.ds(b0, BPT), :], dst_buf)
    plsc.subcore_barrier()

    def body(k, _):
        for j in range(8):
            pltpu.async_copy(ones_buf, deg_shared.at[dst_buf.at[8 * k + j]],
                             sem, add=True)
        for j in range(8):
            pltpu.make_async_copy(deg_out.at[0, pl.ds(0, 128)], ones_buf,
                                  sem).wait()
        return 0
    lax.fori_loop(0, BPT // 8, body, 0)

    plsc.subcore_barrier()
    pltpu.sync_copy(deg_shared.at[pl.ds(s * RPT, RPT)], row_buf)
    pltpu.sync_copy(row_buf, deg_out.at[c, pl.ds(s * RPT, RPT)])


@functools.partial(
    pl.kernel,
    out_type=jax.ShapeDtypeStruct((NC, NP, D), jnp.float32),
    mesh=plsc.VectorSubcoreMesh(**_SC_MESH),
    scratch_types=[
        pltpu.VMEM((2, HB, 128), jnp.int32),
        pltpu.VMEM((128, D), jnp.float32),
        pltpu.VMEM((128, D), jnp.float32),
        pltpu.VMEM_SHARED((NP, D), jnp.float32),
        pltpu.SemaphoreType.DMA,
        pltpu.SemaphoreType.DMA,
        pltpu.SemaphoreType.DMA,
        pltpu.SemaphoreType.DMA,
    ],
    compiler_params=pltpu.CompilerParams(use_tc_tiling_on_sc=False),
)
def _sc_prop(g_hbm, ei_hbm, acc_out, idx_buf, rows_a, rows_b,
             acc_shared, sem_a, sem_b, sem_sa, sem_sb):
    c = lax.axis_index("c")
    s = lax.axis_index("s")

    zeros16 = jnp.zeros((16,), jnp.float32)

    def zbody(i, _):
        for j in range(D // 16):
            rows_a[i, pl.ds(j * 16, 16)] = zeros16
        return 0
    lax.fori_loop(0, 128, zbody, 0)
    for k in range(RPT // 128):
        pltpu.sync_copy(rows_a, acc_shared.at[pl.ds(s * RPT + k * 128, 128), :])
    plsc.subcore_barrier()

    b0 = c * (EB // NC) + s * BPT

    def _wait64k(sem):
        pltpu.make_async_copy(g_hbm.at[pl.ds(0, 128), :], rows_a, sem).wait()

    for half in range(2):
        pltpu.sync_copy(ei_hbm.at[:, pl.ds(b0 + half * HB, HB), :], idx_buf)
        pltpu.async_copy(g_hbm.at[idx_buf.at[0, 0]], rows_a, sem_a)

        def step(t, _):
            e = 2 * t
            pltpu.async_copy(g_hbm.at[idx_buf.at[0, e + 1]], rows_b, sem_b)
            _wait64k(sem_a)
            pltpu.sync_copy(rows_a, acc_shared.at[idx_buf.at[1, e]],
                            add=True)

            @pl.when(e + 2 < HB)
            def _():
                pltpu.async_copy(g_hbm.at[idx_buf.at[0, e + 2]], rows_a,
                                 sem_a)
            _wait64k(sem_b)
            pltpu.sync_copy(rows_b, acc_shared.at[idx_buf.at[1, e + 1]],
                            add=True)
            return 0

        lax.fori_loop(0, HB // 2, step, 0)

    plsc.subcore_barrier()
    for k in range(RPT // 128):
        sl = pl.ds(s * RPT + k * 128, 128)
        pltpu.sync_copy(acc_shared.at[sl, :], acc_out.at[c, sl, :])


BM = 512
_GRID = NP // BM


def _dinv_col(degb):
    ones = jnp.ones((NC, 1), jnp.float32)
    deg = lax.dot_general(degb, ones, (((0,), (0,)), ((), ())),
                          preferred_element_type=jnp.float32)
    return lax.rsqrt(deg + 1.0)


def _tc1_body(x_ref, w1_ref, deg_ref, g_ref):
    dinv = _dinv_col(deg_ref[...])
    hw = jnp.dot(x_ref[...], w1_ref[...], preferred_element_type=jnp.float32)
    g_ref[...] = hw * dinv


def _tc2_body(acc_ref, g_ref, deg_ref, b1_ref, w2_ref, g2_ref):
    dinv = _dinv_col(deg_ref[...])
    h = jnp.maximum(dinv * (acc_ref[0] + acc_ref[1] + g_ref[...])
                    + b1_ref[...], 0.0)
    g2_ref[...] = jnp.dot(h, w2_ref[...],
                          preferred_element_type=jnp.float32) * dinv


def _tc3_body(acc_ref, g_ref, deg_ref, b2_ref, wp_ref, bp_ref, z_ref):
    dinv = _dinv_col(deg_ref[...])
    h = jnp.maximum(dinv * (acc_ref[0] + acc_ref[1] + g_ref[...])
                    + b2_ref[...], 0.0)
    z_ref[...] = jnp.dot(h, wp_ref[...],
                         preferred_element_type=jnp.float32) + bp_ref[...]


def _spec_rows(cols):
    return pl.BlockSpec((BM, cols), lambda i: (i, 0))


def _spec_acc():
    return pl.BlockSpec((NC, BM, D), lambda i: (0, i, 0))


def _spec_deg():
    return pl.BlockSpec((NC, BM), lambda i: (0, i))


def _spec_full(r, c):
    return pl.BlockSpec((r, c), lambda i: (0, 0))


_tc1 = pl.pallas_call(
    _tc1_body,
    grid=(_GRID,),
    in_specs=[_spec_rows(D), _spec_full(D, D), _spec_deg()],
    out_specs=_spec_rows(D),
    out_shape=jax.ShapeDtypeStruct((NP, D), jnp.float32),
)

_tc2 = pl.pallas_call(
    _tc2_body,
    grid=(_GRID,),
    in_specs=[_spec_acc(), _spec_rows(D), _spec_deg(),
              _spec_full(1, D), _spec_full(D, D)],
    out_specs=_spec_rows(D),
    out_shape=jax.ShapeDtypeStruct((NP, D), jnp.float32),
)

_tc3 = pl.pallas_call(
    _tc3_body,
    grid=(_GRID,),
    in_specs=[_spec_acc(), _spec_rows(D), _spec_deg(),
              _spec_full(1, D), _spec_full(D, OUT), _spec_full(1, OUT)],
    out_specs=pl.BlockSpec((BM, OUT), lambda i: (i, 0)),
    out_shape=jax.ShapeDtypeStruct((N, OUT), jnp.float32),
)

_PAD_EDGES = np.ascontiguousarray(
    np.broadcast_to(10000 + (np.arange(EP - E, dtype=np.int32) % (NP - N)),
                    (2, EP - E)))


def kernel(x, edge_index, W1, b1, W2, b2, Wp, bp):
    ei = edge_index.astype(jnp.int32)
    ei_p = jnp.concatenate([ei, _PAD_EDGES], axis=1).reshape(2, EB, 128)

    deg_parts = _sc_degree(ei_p)
    g1 = _tc1(x, W1, deg_parts)
    acc1 = _sc_prop(g1, ei_p)
    g2 = _tc2(acc1, g1, deg_parts, b1.reshape(1, D), W2)
    acc2 = _sc_prop(g2, ei_p)
    return _tc3(acc2, g2, deg_parts, b2.reshape(1, D), Wp, bp.reshape(1, OUT))


# speedup vs baseline: 1.0695x; 1.0695x over previous
import functools

import numpy as np

import jax
import jax.numpy as jnp
from jax import lax
from jax.experimental import pallas as pl
from jax.experimental.pallas import tpu as pltpu
from jax.experimental.pallas import tpu_sc as plsc

N = 10000
NP = 10240
E = 320000
EP = 327680
EB = EP // 128
D = 128
OUT = 64
NC = 2
NS = 16
RPT = NP // NS
BPT = EB // (NC * NS)
HB = BPT // 2

_SC_MESH = dict(core_axis_name="c", subcore_axis_name="s",
                num_cores=NC, num_subcores=NS)


@functools.partial(
    pl.kernel,
    out_type=jax.ShapeDtypeStruct((NC, NP), jnp.float32),
    mesh=plsc.VectorSubcoreMesh(**_SC_MESH),
    scratch_types=[
        pltpu.VMEM((BPT, 128), jnp.int32),
        pltpu.VMEM((128,), jnp.float32),
        pltpu.VMEM((RPT,), jnp.float32),
        pltpu.VMEM_SHARED((NP,), jnp.float32),
        pltpu.SemaphoreType.DMA,
    ],
)
def _sc_degree(ei_hbm, deg_out, dst_buf, ones_buf, row_buf, deg_shared, sem):
    c = lax.axis_index("c")
    s = lax.axis_index("s")
    ones16 = jnp.ones((16,), jnp.float32)
    zeros16 = jnp.zeros((16,), jnp.float32)
    for i in range(8):
        ones_buf[pl.ds(i * 16, 16)] = ones16

    def zbody(i, _):
        row_buf[pl.ds(i * 16, 16)] = zeros16
        return 0
    lax.fori_loop(0, RPT // 16, zbody, 0)
    pltpu.sync_copy(row_buf, deg_shared.at[pl.ds(s * RPT, RPT)])

    b0 = c * (EB // NC) + s * BPT
    pltpu.sync_copy(ei_hbm.at[1, pl.ds(b0, BPT), :], dst_buf)
    plsc.subcore_barrier()

    def body(k, _):
        for j in range(8):
            pltpu.async_copy(ones_buf, deg_shared.at[dst_buf.at[8 * k + j]],
                             sem, add=True)
        for j in range(8):
            pltpu.make_async_copy(deg_out.at[0, pl.ds(0, 128)], ones_buf,
                                  sem).wait()
        return 0
    lax.fori_loop(0, BPT // 8, body, 0)

    plsc.subcore_barrier()
    pltpu.sync_copy(deg_shared.at[pl.ds(s * RPT, RPT)], row_buf)
    pltpu.sync_copy(row_buf, deg_out.at[c, pl.ds(s * RPT, RPT)])


@functools.partial(
    pl.kernel,
    out_type=jax.ShapeDtypeStruct((NC, NP, D), jnp.float32),
    mesh=plsc.VectorSubcoreMesh(**_SC_MESH),
    scratch_types=[
        pltpu.VMEM((2, HB, 128), jnp.int32),
        pltpu.VMEM((128, D), jnp.float32),
        pltpu.VMEM((128, D), jnp.float32),
        pltpu.VMEM_SHARED((NP, D), jnp.float32),
        pltpu.SemaphoreType.DMA,
        pltpu.SemaphoreType.DMA,
        pltpu.SemaphoreType.DMA,
        pltpu.SemaphoreType.DMA,
    ],
    compiler_params=pltpu.CompilerParams(use_tc_tiling_on_sc=False),
)
def _sc_prop(g_hbm, ei_hbm, acc_out, idx_buf, rows_a, rows_b,
             acc_shared, sem_a, sem_b, sem_sa, sem_sb):
    c = lax.axis_index("c")
    s = lax.axis_index("s")

    zeros16 = jnp.zeros((16,), jnp.float32)

    def zbody(i, _):
        for j in range(D // 16):
            rows_a[i, pl.ds(j * 16, 16)] = zeros16
        return 0
    lax.fori_loop(0, 128, zbody, 0)
    for k in range(RPT // 128):
        pltpu.sync_copy(rows_a, acc_shared.at[pl.ds(s * RPT + k * 128, 128), :])
    plsc.subcore_barrier()

    b0 = c * (EB // NC) + s * BPT

    def _wait64k(sem):
        pltpu.make_async_copy(g_hbm.at[pl.ds(0, 128), :], rows_a, sem).wait()

    for half in range(2):
        pltpu.sync_copy(ei_hbm.at[:, pl.ds(b0 + half * HB, HB), :], idx_buf)
        pltpu.async_copy(g_hbm.at[idx_buf.at[0, 0]], rows_a, sem_a)

        def step(t, _):
            e = 2 * t
            pltpu.async_copy(g_hbm.at[idx_buf.at[0, e + 1]], rows_b, sem_b)
            _wait64k(sem_a)
            pltpu.sync_copy(rows_a, acc_shared.at[idx_buf.at[1, e]],
                            add=True)

            @pl.when(e + 2 < HB)
            def _():
                pltpu.async_copy(g_hbm.at[idx_buf.at[0, e + 2]], rows_a,
                                 sem_a)
            _wait64k(sem_b)
            pltpu.sync_copy(rows_b, acc_shared.at[idx_buf.at[1, e + 1]],
                            add=True)
            return 0

        lax.fori_loop(0, HB // 2, step, 0)

    plsc.subcore_barrier()
    for k in range(RPT // 128):
        sl = pl.ds(s * RPT + k * 128, 128)
        pltpu.sync_copy(acc_shared.at[sl, :], acc_out.at[c, sl, :])


BM = 1024
_GRID = NP // BM


def _dinv_col(degb):
    ones = jnp.ones((NC, 1), jnp.float32)
    deg = lax.dot_general(degb, ones, (((0,), (0,)), ((), ())),
                          preferred_element_type=jnp.float32)
    return lax.rsqrt(deg + 1.0)


def _tc1_body(x_ref, w1_ref, deg_ref, g_ref):
    dinv = _dinv_col(deg_ref[...])
    hw = jnp.dot(x_ref[...], w1_ref[...], preferred_element_type=jnp.float32)
    g_ref[...] = hw * dinv


def _tc2_body(acc_ref, g_ref, deg_ref, b1_ref, w2_ref, g2_ref):
    dinv = _dinv_col(deg_ref[...])
    h = jnp.maximum(dinv * (acc_ref[0] + acc_ref[1] + g_ref[...])
                    + b1_ref[...], 0.0)
    g2_ref[...] = jnp.dot(h, w2_ref[...],
                          preferred_element_type=jnp.float32) * dinv


def _tc3_body(acc_ref, g_ref, deg_ref, b2_ref, wp_ref, bp_ref, z_ref):
    dinv = _dinv_col(deg_ref[...])
    h = jnp.maximum(dinv * (acc_ref[0] + acc_ref[1] + g_ref[...])
                    + b2_ref[...], 0.0)
    z_ref[...] = jnp.dot(h, wp_ref[...],
                         preferred_element_type=jnp.float32) + bp_ref[...]


def _spec_rows(cols):
    return pl.BlockSpec((BM, cols), lambda i: (i, 0))


def _spec_acc():
    return pl.BlockSpec((NC, BM, D), lambda i: (0, i, 0))


def _spec_deg():
    return pl.BlockSpec((NC, BM), lambda i: (0, i))


def _spec_full(r, c):
    return pl.BlockSpec((r, c), lambda i: (0, 0))


_tc1 = pl.pallas_call(
    _tc1_body,
    grid=(_GRID,),
    in_specs=[_spec_rows(D), _spec_full(D, D), _spec_deg()],
    out_specs=_spec_rows(D),
    out_shape=jax.ShapeDtypeStruct((NP, D), jnp.float32),
)

_tc2 = pl.pallas_call(
    _tc2_body,
    grid=(_GRID,),
    in_specs=[_spec_acc(), _spec_rows(D), _spec_deg(),
              _spec_full(1, D), _spec_full(D, D)],
    out_specs=_spec_rows(D),
    out_shape=jax.ShapeDtypeStruct((NP, D), jnp.float32),
)

_tc3 = pl.pallas_call(
    _tc3_body,
    grid=(_GRID,),
    in_specs=[_spec_acc(), _spec_rows(D), _spec_deg(),
              _spec_full(1, D), _spec_full(D, OUT), _spec_full(1, OUT)],
    out_specs=pl.BlockSpec((BM, OUT), lambda i: (i, 0)),
    out_shape=jax.ShapeDtypeStruct((N, OUT), jnp.float32),
)

_PAD_EDGES = np.ascontiguousarray(
    np.broadcast_to(10000 + (np.arange(EP - E, dtype=np.int32) % (NP - N)),
                    (2, EP - E)))


def kernel(x, edge_index, W1, b1, W2, b2, Wp, bp):
    ei = edge_index.astype(jnp.int32)
    ei_p = jnp.concatenate([ei, _PAD_EDGES], axis=1).reshape(2, EB, 128)

    deg_parts = _sc_degree(ei_p)
    g1 = _tc1(x, W1, deg_parts)
    acc1 = _sc_prop(g1, ei_p)
    g2 = _tc2(acc1, g1, deg_parts, b1.reshape(1, D), W2)
    acc2 = _sc_prop(g2, ei_p)
    return _tc3(acc2, g2, deg_parts, b2.reshape(1, D), Wp, bp.reshape(1, OUT))


# speedup vs baseline: 1.0843x; 1.0138x over previous
import functools

import numpy as np

import jax
import jax.numpy as jnp
from jax import lax
from jax.experimental import pallas as pl
from jax.experimental.pallas import tpu as pltpu
from jax.experimental.pallas import tpu_sc as plsc

N = 10000
NP = 10240
E = 320000
EP = 327680
EB = EP // 128
D = 128
OUT = 64
NC = 2
NS = 16
RPT = NP // NS
BPT = EB // (NC * NS)
HB = BPT // 2

_SC_MESH = dict(core_axis_name="c", subcore_axis_name="s",
                num_cores=NC, num_subcores=NS)


@functools.partial(
    pl.kernel,
    out_type=jax.ShapeDtypeStruct((NC, NP), jnp.float32),
    mesh=plsc.VectorSubcoreMesh(**_SC_MESH),
    scratch_types=[
        pltpu.VMEM((BPT, 128), jnp.int32),
        pltpu.VMEM((128,), jnp.float32),
        pltpu.VMEM((RPT,), jnp.float32),
        pltpu.VMEM_SHARED((NP,), jnp.float32),
        pltpu.SemaphoreType.DMA,
    ],
)
def _sc_degree(ei_hbm, deg_out, dst_buf, ones_buf, row_buf, deg_shared, sem):
    c = lax.axis_index("c")
    s = lax.axis_index("s")
    ones16 = jnp.ones((16,), jnp.float32)
    zeros16 = jnp.zeros((16,), jnp.float32)
    for i in range(8):
        ones_buf[pl.ds(i * 16, 16)] = ones16

    def zbody(i, _):
        row_buf[pl.ds(i * 16, 16)] = zeros16
        return 0
    lax.fori_loop(0, RPT // 16, zbody, 0)
    pltpu.sync_copy(row_buf, deg_shared.at[pl.ds(s * RPT, RPT)])

    b0 = c * (EB // NC) + s * BPT
    pltpu.sync_copy(ei_hbm.at[1, pl.ds(b0, BPT), :], dst_buf)
    plsc.subcore_barrier()

    def body(k, _):
        for j in range(8):
            pltpu.async_copy(ones_buf, deg_shared.at[dst_buf.at[8 * k + j]],
                             sem, add=True)
        for j in range(8):
            pltpu.make_async_copy(deg_out.at[0, pl.ds(0, 128)], ones_buf,
                                  sem).wait()
        return 0
    lax.fori_loop(0, BPT // 8, body, 0)

    plsc.subcore_barrier()
    pltpu.sync_copy(deg_shared.at[pl.ds(s * RPT, RPT)], row_buf)
    pltpu.sync_copy(row_buf, deg_out.at[c, pl.ds(s * RPT, RPT)])


@functools.partial(
    pl.kernel,
    out_type=jax.ShapeDtypeStruct((NC, NP, D), jnp.float32),
    mesh=plsc.VectorSubcoreMesh(**_SC_MESH),
    scratch_types=[
        pltpu.VMEM((2, 2 * HB, 64), jnp.int32),
        pltpu.VMEM((4, 64, D), jnp.float32),
        pltpu.VMEM_SHARED((NP, D), jnp.float32),
        [pltpu.SemaphoreType.DMA] * 4,
        [pltpu.SemaphoreType.DMA] * 4,
    ],
    compiler_params=pltpu.CompilerParams(use_tc_tiling_on_sc=False),
)
def _sc_prop(g_hbm, ei_hbm, acc_out, idx_buf, rows, acc_shared,
             gsems, ssems):
    c = lax.axis_index("c")
    s = lax.axis_index("s")

    zeros16 = jnp.zeros((16,), jnp.float32)

    def zbody(i, _):
        for j in range(D // 16):
            rows[0, i, pl.ds(j * 16, 16)] = zeros16
            rows[1, i, pl.ds(j * 16, 16)] = zeros16
        return 0
    lax.fori_loop(0, 64, zbody, 0)
    for k in range(RPT // 128):
        pltpu.sync_copy(rows.at[0],
                        acc_shared.at[pl.ds(s * RPT + k * 128, 64), :])
        pltpu.sync_copy(rows.at[1],
                        acc_shared.at[pl.ds(s * RPT + k * 128 + 64, 64), :])
    plsc.subcore_barrier()

    b0 = 2 * (c * (EB // NC) + s * BPT)
    NBH = 2 * HB

    def _wait32k(sem):
        pltpu.make_async_copy(g_hbm.at[pl.ds(0, 64), :], rows.at[0],
                              sem).wait()

    def _gather(k, buf):
        pltpu.async_copy(g_hbm.at[idx_buf.at[0, k]], rows.at[buf],
                         gsems[buf])

    def _scatter(k, buf):
        pltpu.async_copy(rows.at[buf], acc_shared.at[idx_buf.at[1, k]],
                         ssems[buf], add=True)

    for half in range(2):
        pltpu.sync_copy(ei_hbm.at[:, pl.ds(b0 + half * NBH, NBH), :],
                        idx_buf)
        _gather(0, 0)
        _gather(1, 1)

        def quad(t, _):
            for q in range(4):
                k = 4 * t + q
                buf = q
                nxt = (q + 2) % 4

                @pl.when(k >= 2)
                def _():
                    _wait32k(ssems[nxt])

                @pl.when(k + 2 < NBH)
                def _():
                    pltpu.async_copy(g_hbm.at[idx_buf.at[0, k + 2]],
                                     rows.at[nxt], gsems[nxt])

                _wait32k(gsems[buf])
                _scatter(k, buf)
            return 0

        lax.fori_loop(0, NBH // 4, quad, 0)
        _wait32k(ssems[(NBH - 2) % 4])
        _wait32k(ssems[(NBH - 1) % 4])

    plsc.subcore_barrier()
    for k in range(RPT // 128):
        sl = pl.ds(s * RPT + k * 128, 128)
        pltpu.sync_copy(acc_shared.at[sl, :], acc_out.at[c, sl, :])


BM = 1024
_GRID = NP // BM


def _dinv_col(degb):
    ones = jnp.ones((NC, 1), jnp.float32)
    deg = lax.dot_general(degb, ones, (((0,), (0,)), ((), ())),
                          preferred_element_type=jnp.float32)
    return lax.rsqrt(deg + 1.0)


def _tc1_body(x_ref, w1_ref, deg_ref, g_ref):
    dinv = _dinv_col(deg_ref[...])
    hw = jnp.dot(x_ref[...], w1_ref[...], preferred_element_type=jnp.float32)
    g_ref[...] = hw * dinv


def _tc2_body(acc_ref, g_ref, deg_ref, b1_ref, w2_ref, g2_ref):
    dinv = _dinv_col(deg_ref[...])
    h = jnp.maximum(dinv * (acc_ref[0] + acc_ref[1] + g_ref[...])
                    + b1_ref[...], 0.0)
    g2_ref[...] = jnp.dot(h, w2_ref[...],
                          preferred_element_type=jnp.float32) * dinv


def _tc3_body(acc_ref, g_ref, deg_ref, b2_ref, wp_ref, bp_ref, z_ref):
    dinv = _dinv_col(deg_ref[...])
    h = jnp.maximum(dinv * (acc_ref[0] + acc_ref[1] + g_ref[...])
                    + b2_ref[...], 0.0)
    z_ref[...] = jnp.dot(h, wp_ref[...],
                         preferred_element_type=jnp.float32) + bp_ref[...]


def _spec_rows(cols):
    return pl.BlockSpec((BM, cols), lambda i: (i, 0))


def _spec_acc():
    return pl.BlockSpec((NC, BM, D), lambda i: (0, i, 0))


def _spec_deg():
    return pl.BlockSpec((NC, BM), lambda i: (0, i))


def _spec_full(r, c):
    return pl.BlockSpec((r, c), lambda i: (0, 0))


_tc1 = pl.pallas_call(
    _tc1_body,
    grid=(_GRID,),
    in_specs=[_spec_rows(D), _spec_full(D, D), _spec_deg()],
    out_specs=_spec_rows(D),
    out_shape=jax.ShapeDtypeStruct((NP, D), jnp.float32),
)

_tc2 = pl.pallas_call(
    _tc2_body,
    grid=(_GRID,),
    in_specs=[_spec_acc(), _spec_rows(D), _spec_deg(),
              _spec_full(1, D), _spec_full(D, D)],
    out_specs=_spec_rows(D),
    out_shape=jax.ShapeDtypeStruct((NP, D), jnp.float32),
)

_tc3 = pl.pallas_call(
    _tc3_body,
    grid=(_GRID,),
    in_specs=[_spec_acc(), _spec_rows(D), _spec_deg(),
              _spec_full(1, D), _spec_full(D, OUT), _spec_full(1, OUT)],
    out_specs=pl.BlockSpec((BM, OUT), lambda i: (i, 0)),
    out_shape=jax.ShapeDtypeStruct((N, OUT), jnp.float32),
)

_PAD_EDGES = np.ascontiguousarray(
    np.broadcast_to(10000 + (np.arange(EP - E, dtype=np.int32) % (NP - N)),
                    (2, EP - E)))


def kernel(x, edge_index, W1, b1, W2, b2, Wp, bp):
    ei = edge_index.astype(jnp.int32)
    ei_pad = jnp.concatenate([ei, _PAD_EDGES], axis=1)
    ei_p = ei_pad.reshape(2, EB, 128)
    ei_p64 = ei_pad.reshape(2, 2 * EB, 64)

    deg_parts = _sc_degree(ei_p)
    g1 = _tc1(x, W1, deg_parts)
    acc1 = _sc_prop(g1, ei_p64)
    g2 = _tc2(acc1, g1, deg_parts, b1.reshape(1, D), W2)
    acc2 = _sc_prop(g2, ei_p64)
    return _tc3(acc2, g2, deg_parts, b2.reshape(1, D), Wp, bp.reshape(1, OUT))
